# Initial kernel scaffold; baseline (speedup 1.0000x reference)
import jax
import jax.numpy as jnp
from jax.experimental import pallas as pl


def kernel(feat, edge_index, W_neigh, W_self, b_self):
    raise NotImplementedError("write your pallas kernel here")


# speedup vs baseline: 7.0066x; 7.0066x over previous
import functools

import jax
import jax.numpy as jnp
from jax import lax
from jax.experimental import pallas as pl
from jax.experimental.pallas import tpu as pltpu
from jax.experimental.pallas import tpu_sc as plsc

N_NODES = 10000
D = 128
DH = D // 2
E_EDGES = 320000
NC = 2
NS = 16
L = 16
CSZ = 128
CH = 158
EPW = CH * CSZ
EPAD = EPW * NS
NPAD = 10240
RPT = NPAD // NS

_mesh = plsc.VectorSubcoreMesh(core_axis_name="c", subcore_axis_name="s")


@functools.partial(
    pl.kernel,
    mesh=_mesh,
    compiler_params=pltpu.CompilerParams(use_tc_tiling_on_sc=False),
    out_type=[
        jax.ShapeDtypeStruct((NC, NPAD, DH), jnp.float32),
        jax.ShapeDtypeStruct((NPAD, L), jnp.float32),
    ],
    scratch_types=[
        pltpu.VMEM((CH, CSZ), jnp.int32),
        pltpu.VMEM((CH, CSZ), jnp.int32),
        pltpu.VMEM((CSZ, DH), jnp.float32),
        pltpu.VMEM((CSZ, DH), jnp.float32),
        pltpu.VMEM((CSZ, L), jnp.float32),
        pltpu.VMEM_SHARED((NPAD, DH), jnp.float32),
        pltpu.VMEM_SHARED((NPAD, L), jnp.float32),
        pltpu.SemaphoreType.DMA,
        pltpu.SemaphoreType.DMA,
    ],
)
def _sc_aggregate(feat2_hbm, src_hbm, dst_hbm, z64_hbm, z16_hbm, ones_hbm,
                  acc_out, deg_out, src_v, dst_v, buf0, buf1, ones_v,
                  acc_sh, deg_sh, sem0, sem1):
    cid = lax.axis_index("c")
    sid = lax.axis_index("s")
    table = feat2_hbm.at[cid]

    pltpu.sync_copy(src_hbm.at[sid], src_v)
    pltpu.sync_copy(dst_hbm.at[sid], dst_v)
    pltpu.sync_copy(ones_hbm, ones_v)

    base = sid * RPT
    pltpu.sync_copy(z64_hbm, acc_sh.at[pl.ds(base, RPT)])
    pltpu.sync_copy(z16_hbm, deg_sh.at[pl.ds(base, RPT)])
    plsc.subcore_barrier()

    def g_start(j, buf, sem):
        pltpu.async_copy(table.at[src_v.at[j]], buf, sem)

    def g_wait(j, buf, sem):
        pltpu.make_async_copy(table.at[src_v.at[j]], buf, sem).wait()

    def scat(j, buf):
        pltpu.sync_copy(buf, acc_sh.at[dst_v.at[j]], add=True)
        pltpu.sync_copy(ones_v, deg_sh.at[dst_v.at[j]], add=True)

    g_start(0, buf0, sem0)

    @pl.loop(0, CH - 2, step=2)
    def _(i):
        g_wait(i, buf0, sem0)
        g_start(i + 1, buf1, sem1)
        scat(i, buf0)
        g_wait(i + 1, buf1, sem1)
        g_start(i + 2, buf0, sem0)
        scat(i + 1, buf1)

    g_wait(CH - 2, buf0, sem0)
    g_start(CH - 1, buf1, sem1)
    scat(CH - 2, buf0)
    g_wait(CH - 1, buf1, sem1)
    scat(CH - 1, buf1)

    plsc.subcore_barrier()
    pltpu.sync_copy(acc_sh.at[pl.ds(base, RPT)], acc_out.at[cid, pl.ds(base, RPT)])

    @pl.when(cid == 0)
    def _():
        pltpu.sync_copy(deg_sh.at[pl.ds(base, RPT)], deg_out.at[pl.ds(base, RPT)])


_RB = 2000


def _tc_body(feat_ref, acc_ref, deg_ref, wn_ref, ws_ref, b_ref, o_ref):
    acc = jnp.concatenate([acc_ref[0], acc_ref[1]], axis=-1)
    deg = jnp.maximum(deg_ref[:, 0:1], 1.0)
    h_neigh = acc / deg
    o_ref[...] = (
        jnp.dot(h_neigh, wn_ref[...], preferred_element_type=jnp.float32)
        + jnp.dot(feat_ref[...], ws_ref[...], preferred_element_type=jnp.float32)
        + b_ref[...]
    )


def _tc_combine(feat, acc, deg, wn_t, ws_t, b_row):
    return pl.pallas_call(
        _tc_body,
        grid=(N_NODES // _RB,),
        in_specs=[
            pl.BlockSpec((_RB, D), lambda r: (r, 0)),
            pl.BlockSpec((NC, _RB, DH), lambda r: (0, r, 0)),
            pl.BlockSpec((_RB, L), lambda r: (r, 0)),
            pl.BlockSpec((D, D), lambda r: (0, 0)),
            pl.BlockSpec((D, D), lambda r: (0, 0)),
            pl.BlockSpec((1, D), lambda r: (0, 0)),
        ],
        out_specs=pl.BlockSpec((_RB, D), lambda r: (r, 0)),
        out_shape=jax.ShapeDtypeStruct((N_NODES, D), jnp.float32),
    )(feat, acc, deg, wn_t, ws_t, b_row)


@jax.jit
def kernel(feat, edge_index, W_neigh, W_self, b_self):
    src = edge_index[0]
    dst = edge_index[1]
    pad = EPAD - E_EDGES
    src_p = jnp.concatenate([src, jnp.zeros((pad,), jnp.int32)])
    dst_p = jnp.concatenate([dst, jnp.full((pad,), N_NODES, jnp.int32)])
    src_t = src_p.reshape(NS, CH, CSZ)
    dst_t = dst_p.reshape(NS, CH, CSZ)
    feat2 = jnp.stack([feat[:, :DH], feat[:, DH:]])
    z64 = jnp.zeros((RPT, DH), jnp.float32)
    z16 = jnp.zeros((RPT, L), jnp.float32)
    ones16 = jnp.ones((CSZ, L), jnp.float32)
    acc, deg = _sc_aggregate(feat2, src_t, dst_t, z64, z16, ones16)
    return _tc_combine(feat, acc, deg, W_neigh.T, W_self.T,
                       b_self.reshape(1, D))
